# baseline (speedup 1.0000x reference)
import functools

import jax
import jax.numpy as jnp
from jax import lax
from jax.experimental import pallas as pl
from jax.experimental.pallas import tpu as pltpu

_B = 64
_HH = 32
_STEPS = 10


def _one_type(x, seg_row, W_ref, t, acc_ref):
    Wg = W_ref[t, :128, _HH:_HH + 1]
    l_row = lax.dot_general(Wg, x, (((0,), (1,)), ((), ())),
                            preferred_element_type=jnp.float32,
                            precision=lax.Precision.DEFAULT)
    e_row = jnp.exp(l_row)
    r = seg_row.shape[1]
    iota_b = lax.broadcasted_iota(jnp.int32, (_B, r), 0)
    onehot_e = jnp.where(jnp.broadcast_to(seg_row, (_B, r)) == iota_b,
                         jnp.broadcast_to(e_row, (_B, r)), 0.0)
    acc_ref[:, :128] += jnp.dot(onehot_e, x,
                                preferred_element_type=jnp.float32,
                                precision=lax.Precision.DEFAULT)
    acc_ref[:, 128:129] += jnp.sum(onehot_e, axis=1, keepdims=True)


def _fin(acc_ref, W_ref, t, out_ref):
    Wf = W_ref[t, :128, :_HH]
    num = jnp.dot(acc_ref[:, :128], Wf,
                  preferred_element_type=jnp.float32)
    den = acc_ref[:, 128:129]
    bf = W_ref[t, 128:129, :_HH]
    out_ref[:, t * _HH:(t + 1) * _HH] = jnp.where(den > 0.0,
                                                  num / den + bf, 0.0)


def _body(segw_ref, segt_ref, segd_ref, xw_ref, xt_ref, xd_ref, W_ref,
          out_ref, accw_ref, acct_ref, accd_ref):
    i = pl.program_id(0)

    @pl.when(i == 0)
    def _init():
        accw_ref[...] = jnp.zeros_like(accw_ref)
        acct_ref[...] = jnp.zeros_like(acct_ref)
        accd_ref[...] = jnp.zeros_like(accd_ref)

    _one_type(xw_ref[...], segw_ref[0], W_ref, 0, accw_ref)
    _one_type(xt_ref[...], segt_ref[0], W_ref, 1, acct_ref)
    _one_type(xd_ref[...], segd_ref[0], W_ref, 2, accd_ref)

    @pl.when(i == _STEPS - 1)
    def _finish():
        out_ref[:, 96:128] = jnp.zeros((_B, 32), jnp.float32)
        _fin(accw_ref, W_ref, 0, out_ref)
        _fin(acct_ref, W_ref, 1, out_ref)
        _fin(accd_ref, W_ref, 2, out_ref)


def _pack(Wf, Wg, bf):
    top = jnp.concatenate([Wf, Wg], axis=1)
    bot = jnp.concatenate([bf.reshape(1, _HH),
                           jnp.zeros((1, 1), jnp.float32)], axis=1)
    return jnp.concatenate([top, bot], axis=0)


def kernel(feat_word, feat_topic, feat_doc, seg_word, seg_topic, seg_doc,
           W_feat_word, b_feat_word, W_gate_word, b_gate_word,
           W_feat_topic, b_feat_topic, W_gate_topic, b_gate_topic,
           W_feat_doc, b_feat_doc, W_gate_doc, b_gate_doc):
    rw = feat_word.shape[0] // _STEPS
    rt = feat_topic.shape[0] // _STEPS
    rd = feat_doc.shape[0] // _STEPS
    segw = seg_word.astype(jnp.int32).reshape(_STEPS, 1, rw)
    segt = seg_topic.astype(jnp.int32).reshape(_STEPS, 1, rt)
    segd = seg_doc.astype(jnp.int32).reshape(_STEPS, 1, rd)
    Wall = jnp.stack([_pack(W_feat_word, W_gate_word, b_feat_word),
                      _pack(W_feat_topic, W_gate_topic, b_feat_topic),
                      _pack(W_feat_doc, W_gate_doc, b_feat_doc)])
    out = pl.pallas_call(
        _body,
        grid=(_STEPS,),
        in_specs=[
            pl.BlockSpec((1, 1, rw), lambda i: (i, 0, 0)),
            pl.BlockSpec((1, 1, rt), lambda i: (i, 0, 0)),
            pl.BlockSpec((1, 1, rd), lambda i: (i, 0, 0)),
            pl.BlockSpec((rw, 128), lambda i: (i, 0)),
            pl.BlockSpec((rt, 128), lambda i: (i, 0)),
            pl.BlockSpec((rd, 128), lambda i: (i, 0)),
            pl.BlockSpec((3, 129, 33), lambda i: (0, 0, 0)),
        ],
        out_specs=pl.BlockSpec((_B, 128), lambda i: (0, 0)),
        out_shape=jax.ShapeDtypeStruct((_B, 128), jnp.float32),
        scratch_shapes=[
            pltpu.VMEM((_B, 129), jnp.float32),
            pltpu.VMEM((_B, 129), jnp.float32),
            pltpu.VMEM((_B, 129), jnp.float32),
        ],
    )(segw, segt, segd, feat_word, feat_topic, feat_doc, Wall)
    return (out[:, :_HH], out[:, _HH:2 * _HH], out[:, 2 * _HH:3 * _HH])


# speedup vs baseline: 1.0270x; 1.0270x over previous
import functools

import jax
import jax.numpy as jnp
from jax import lax
from jax.experimental import pallas as pl
from jax.experimental.pallas import tpu as pltpu

_B = 64
_HH = 32
_STEPS = 5


def _one_type(x, seg_row, W_ref, t, acc_ref):
    Wg = W_ref[t, :128, _HH:_HH + 1]
    l_row = lax.dot_general(Wg, x, (((0,), (1,)), ((), ())),
                            preferred_element_type=jnp.float32,
                            precision=lax.Precision.DEFAULT)
    e_row = jnp.exp(l_row)
    r = seg_row.shape[1]
    iota_b = lax.broadcasted_iota(jnp.int32, (_B, r), 0)
    onehot_e = jnp.where(jnp.broadcast_to(seg_row, (_B, r)) == iota_b,
                         jnp.broadcast_to(e_row, (_B, r)), 0.0)
    acc_ref[:, :128] += jnp.dot(onehot_e, x,
                                preferred_element_type=jnp.float32,
                                precision=lax.Precision.DEFAULT)
    acc_ref[:, 128:129] += jnp.sum(onehot_e, axis=1, keepdims=True)


def _fin(acc_ref, W_ref, t, out_ref):
    Wf = W_ref[t, :128, :_HH]
    num = jnp.dot(acc_ref[:, :128], Wf,
                  preferred_element_type=jnp.float32)
    den = acc_ref[:, 128:129]
    bf = W_ref[t, 128:129, :_HH]
    out_ref[:, t * _HH:(t + 1) * _HH] = jnp.where(den > 0.0,
                                                  num / den + bf, 0.0)


def _body(segw_ref, segt_ref, segd_ref, xw_ref, xt_ref, xd_ref, W_ref,
          out_ref, accw_ref, acct_ref, accd_ref):
    i = pl.program_id(0)

    @pl.when(i == 0)
    def _init():
        accw_ref[...] = jnp.zeros_like(accw_ref)
        acct_ref[...] = jnp.zeros_like(acct_ref)
        accd_ref[...] = jnp.zeros_like(accd_ref)

    _one_type(xw_ref[...], segw_ref[0], W_ref, 0, accw_ref)
    _one_type(xt_ref[...], segt_ref[0], W_ref, 1, acct_ref)
    _one_type(xd_ref[...], segd_ref[0], W_ref, 2, accd_ref)

    @pl.when(i == _STEPS - 1)
    def _finish():
        out_ref[:, 96:128] = jnp.zeros((_B, 32), jnp.float32)
        _fin(accw_ref, W_ref, 0, out_ref)
        _fin(acct_ref, W_ref, 1, out_ref)
        _fin(accd_ref, W_ref, 2, out_ref)


def _pack(Wf, Wg, bf):
    top = jnp.concatenate([Wf, Wg], axis=1)
    bot = jnp.concatenate([bf.reshape(1, _HH),
                           jnp.zeros((1, 1), jnp.float32)], axis=1)
    return jnp.concatenate([top, bot], axis=0)


def kernel(feat_word, feat_topic, feat_doc, seg_word, seg_topic, seg_doc,
           W_feat_word, b_feat_word, W_gate_word, b_gate_word,
           W_feat_topic, b_feat_topic, W_gate_topic, b_gate_topic,
           W_feat_doc, b_feat_doc, W_gate_doc, b_gate_doc):
    rw = feat_word.shape[0] // _STEPS
    rt = feat_topic.shape[0] // _STEPS
    rd = feat_doc.shape[0] // _STEPS
    segw = seg_word.astype(jnp.int32).reshape(_STEPS, 1, rw)
    segt = seg_topic.astype(jnp.int32).reshape(_STEPS, 1, rt)
    segd = seg_doc.astype(jnp.int32).reshape(_STEPS, 1, rd)
    Wall = jnp.stack([_pack(W_feat_word, W_gate_word, b_feat_word),
                      _pack(W_feat_topic, W_gate_topic, b_feat_topic),
                      _pack(W_feat_doc, W_gate_doc, b_feat_doc)])
    out = pl.pallas_call(
        _body,
        grid=(_STEPS,),
        in_specs=[
            pl.BlockSpec((1, 1, rw), lambda i: (i, 0, 0)),
            pl.BlockSpec((1, 1, rt), lambda i: (i, 0, 0)),
            pl.BlockSpec((1, 1, rd), lambda i: (i, 0, 0)),
            pl.BlockSpec((rw, 128), lambda i: (i, 0)),
            pl.BlockSpec((rt, 128), lambda i: (i, 0)),
            pl.BlockSpec((rd, 128), lambda i: (i, 0)),
            pl.BlockSpec((3, 129, 33), lambda i: (0, 0, 0)),
        ],
        out_specs=pl.BlockSpec((_B, 128), lambda i: (0, 0)),
        out_shape=jax.ShapeDtypeStruct((_B, 128), jnp.float32),
        scratch_shapes=[
            pltpu.VMEM((_B, 129), jnp.float32),
            pltpu.VMEM((_B, 129), jnp.float32),
            pltpu.VMEM((_B, 129), jnp.float32),
        ],
    )(segw, segt, segd, feat_word, feat_topic, feat_doc, Wall)
    return (out[:, :_HH], out[:, _HH:2 * _HH], out[:, 2 * _HH:3 * _HH])
